# Initial kernel scaffold; baseline (speedup 1.0000x reference)
import jax
import jax.numpy as jnp
from jax.experimental import pallas as pl


def kernel(query):
    raise NotImplementedError("write your pallas kernel here")


# speedup vs baseline: 1.3743x; 1.3743x over previous
import functools

import jax
import jax.numpy as jnp
from jax import lax
from jax.experimental import pallas as pl
from jax.experimental.pallas import tpu as pltpu
from jax.experimental.pallas import tpu_sc as plsc

N = 100000
PC = 800
NPAD = 102400
CHUNK = NPAD // 16
NV = CHUNK // 16
ROWS = 896
TRASH = PC
PAD_COORD = 1e18
RCOLS = 896
BM = 1000


def _tc_keys(xs, ys, zs):
    def body(x_ref, y_ref, z_ref, o_ref):
        x = x_ref[...]
        y = y_ref[...]
        z = z_ref[...]
        a = jnp.abs(jnp.sqrt(x * x + y * y + z * z) - 0.5)
        o_ref[...] = lax.bitcast_convert_type(a, jnp.int32)

    kr = NPAD // 128
    return pl.pallas_call(
        body,
        out_shape=jax.ShapeDtypeStruct((kr, 128), jnp.int32),
    )(xs.reshape(kr, 128), ys.reshape(kr, 128), zs.reshape(kr, 128))


def _sc_select(xs, ys, zs, keys):
    mesh = plsc.VectorSubcoreMesh(core_axis_name="c", subcore_axis_name="s")

    @functools.partial(
        pl.kernel,
        out_type=jax.ShapeDtypeStruct((PC + 8, 16), jnp.float32),
        mesh=mesh,
        compiler_params=pltpu.CompilerParams(
            needs_layout_passes=False, use_tc_tiling_on_sc=False),
        scratch_types=[
            pltpu.VMEM((CHUNK,), jnp.float32),
            pltpu.VMEM((CHUNK,), jnp.float32),
            pltpu.VMEM((CHUNK,), jnp.float32),
            pltpu.VMEM((CHUNK,), jnp.int32),
            pltpu.VMEM((ROWS, 16), jnp.float32),
            pltpu.VMEM((ROWS, 16), jnp.float32),
            pltpu.VMEM((16,), jnp.int32),
            pltpu.VMEM((16, 16), jnp.int32),
            pltpu.VMEM((7, 128), jnp.int32),
            pltpu.VMEM((7, 128), jnp.int32),
            pltpu.VMEM_SHARED((1, 16, 16), jnp.int32),
            pltpu.SemaphoreType.DMA,
        ],
    )
    def sel(xs_hbm, ys_hbm, zs_hbm, kb_hbm, out_hbm,
            xv, yv, zv, kv, blt, beq, pubv, rbv, ilt, ieq, shared, sem):
        sid = lax.axis_index("s")
        base = sid * CHUNK
        pltpu.sync_copy(xs_hbm.at[pl.ds(base, CHUNK)], xv)
        pltpu.sync_copy(ys_hbm.at[pl.ds(base, CHUNK)], yv)
        pltpu.sync_copy(zs_hbm.at[pl.ds(base, CHUNK)], zv)
        pltpu.sync_copy(kb_hbm.at[pl.ds(base, CHUNK)], kv)
        iota = lax.iota(jnp.int32, 16)

        def publish_sum(slab, vec16):
            del slab
            plsc.subcore_barrier()
            pubv[...] = vec16
            pltpu.sync_copy(pubv, shared.at[0, sid])
            plsc.subcore_barrier()
            pltpu.sync_copy(shared.at[0], rbv)
            tot = rbv[0]
            for r in range(1, 16):
                tot = tot + rbv[r]
            return tot

        def count_le(t):
            def body(i, acc):
                m = kv[pl.ds(i * 16, 16)] <= t
                return acc + jnp.where(m, 1, 0)
            return lax.fori_loop(0, NV, body, jnp.zeros((16,), jnp.int32))

        lo = jnp.int32(0)
        hi = jnp.int32(0x7F800000)
        for _ in range(31):
            mid = lo + ((hi - lo) >> 1)
            cnt = jnp.sum(publish_sum(0, count_le(mid)))
            ok = cnt >= PC
            lo = jnp.where(ok, lo, mid + 1)
            hi = jnp.where(ok, mid, hi)
        kbits = lo

        def cc_body(i, carry):
            olt, oeq = carry
            v = kv[pl.ds(i * 16, 16)]
            x = xv[pl.ds(i * 16, 16)]
            y = yv[pl.ds(i * 16, 16)]
            z = zv[pl.ds(i * 16, 16)]
            mlt = v < kbits
            ilt_ = jnp.where(mlt, 1, 0)
            plt = plsc.cumsum(ilt_)
            rlt = olt + (plt - ilt_)
            c0 = jnp.zeros((16,), jnp.int32)
            plsc.store_scatter(blt, [rlt, c0], x, mask=mlt)
            plsc.store_scatter(blt, [rlt, c0 + 1], y, mask=mlt)
            plsc.store_scatter(blt, [rlt, c0 + 2], z, mask=mlt)
            meq = v == kbits
            ieq_ = jnp.where(meq, 1, 0)
            peq = plsc.cumsum(ieq_)
            req = oeq + (peq - ieq_)
            meqw = meq & (req < ROWS)
            plsc.store_scatter(beq, [req, c0], x, mask=meqw)
            plsc.store_scatter(beq, [req, c0 + 1], y, mask=meqw)
            plsc.store_scatter(beq, [req, c0 + 2], z, mask=meqw)
            return olt + jnp.max(plt), oeq + jnp.max(peq)

        n_lt, n_eq = lax.fori_loop(
            0, NV, cc_body, (jnp.int32(0), jnp.int32(0)))

        v_lt = publish_sum(31, jnp.where(iota == sid, n_lt, 0))
        v_eq = publish_sum(32, jnp.where(iota == sid, n_eq, 0))
        lt_before = jnp.sum(jnp.where(iota < sid, v_lt, 0))
        eq_before = jnp.sum(jnp.where(iota < sid, v_eq, 0))
        total_lt = jnp.sum(v_lt)
        need_eq = PC - total_lt
        take_eq = jnp.clip(need_eq - eq_before, 0, n_eq)
        p_lt = lt_before
        p_eq = total_lt + jnp.minimum(eq_before, need_eq)

        def idx_body(j, _):
            r = j * 16 + iota
            ilt[j >> 3, pl.ds((j & 7) * 16, 16)] = jnp.where(
                r < n_lt, p_lt + r, TRASH)
            ieq[j >> 3, pl.ds((j & 7) * 16, 16)] = jnp.where(
                r < take_eq, p_eq + r, TRASH)
            return 0

        lax.fori_loop(0, ROWS // 16, idx_body, 0)

        for j in range(ROWS // 128):
            @pl.when(n_lt > j * 128)
            def _scat_lt(j=j):
                pltpu.async_copy(
                    blt.at[pl.ds(j * 128, 128)],
                    out_hbm.at[ilt.at[j]], sem).wait()

            @pl.when(take_eq > j * 128)
            def _scat_eq(j=j):
                pltpu.async_copy(
                    beq.at[pl.ds(j * 128, 128)],
                    out_hbm.at[ieq.at[j]], sem).wait()

    return sel(xs, ys, zs, keys)


def _tc_repair(q8, rt):
    def body(q_ref, rt_ref, o_ref):
        q = q_ref[...]
        rt = rt_ref[...]
        g = jnp.dot(q, rt, preferred_element_type=jnp.float32)
        q2 = jnp.sum(q * q, axis=1, keepdims=True)
        r2 = jnp.sum(rt * rt, axis=0, keepdims=True)
        m2 = jnp.min(r2 - 2.0 * g, axis=1, keepdims=True)
        nd = jnp.sqrt(jnp.maximum(q2 + m2, 0.0))
        d = jnp.sqrt(q2) - 0.5
        o_ref[...] = jnp.minimum(nd * jnp.sign(d), d)

    return pl.pallas_call(
        body,
        grid=(N // BM,),
        in_specs=[
            pl.BlockSpec((BM, 8), lambda i: (i, 0)),
            pl.BlockSpec((8, RCOLS), lambda i: (0, 0)),
        ],
        out_specs=pl.BlockSpec((BM, 1), lambda i: (i, 0)),
        out_shape=jax.ShapeDtypeStruct((N, 1), jnp.float32),
    )(q8, rt)


def kernel(query):
    q = query.astype(jnp.float32)
    pad = jnp.full((NPAD - N, 3), PAD_COORD, jnp.float32)
    qp = jnp.concatenate([q, pad], axis=0)
    xs = qp[:, 0]
    ys = qp[:, 1]
    zs = qp[:, 2]
    keys = _tc_keys(xs, ys, zs).reshape(NPAD)
    r16 = _sc_select(xs, ys, zs, keys)
    reps = r16[:PC, :3]
    rt = jnp.zeros((8, RCOLS), jnp.float32)
    rt = rt.at[:3, :PC].set(reps.T)
    rt = rt.at[:3, PC:].set(1e17)
    q8 = jnp.pad(q, ((0, 0), (0, 5)))
    out = _tc_repair(q8, rt)
    return out[:, 0]


# speedup vs baseline: 1.4264x; 1.0379x over previous
import functools

import jax
import jax.numpy as jnp
from jax import lax
from jax.experimental import pallas as pl
from jax.experimental.pallas import tpu as pltpu
from jax.experimental.pallas import tpu_sc as plsc

N = 100000
PC = 800
NPAD = 102400
CHUNK = NPAD // 16
NV = CHUNK // 16
ROWS = 896
TRASH = PC
PAD_COORD = 1e18
RCOLS = 896
BM = 1000


def _tc_keys(xs, ys, zs):
    def body(x_ref, y_ref, z_ref, o_ref):
        x = x_ref[...]
        y = y_ref[...]
        z = z_ref[...]
        a = jnp.abs(jnp.sqrt(x * x + y * y + z * z) - 0.5)
        o_ref[...] = lax.bitcast_convert_type(a, jnp.int32)

    kr = NPAD // 128
    return pl.pallas_call(
        body,
        out_shape=jax.ShapeDtypeStruct((kr, 128), jnp.int32),
    )(xs.reshape(kr, 128), ys.reshape(kr, 128), zs.reshape(kr, 128))


def _sc_select(xs, ys, zs, keys):
    mesh = plsc.VectorSubcoreMesh(
        core_axis_name="c", subcore_axis_name="s", num_cores=1)

    @functools.partial(
        pl.kernel,
        out_type=jax.ShapeDtypeStruct((PC + 8, 16), jnp.float32),
        mesh=mesh,
        compiler_params=pltpu.CompilerParams(
            needs_layout_passes=False, use_tc_tiling_on_sc=False),
        scratch_types=[
            pltpu.VMEM((CHUNK,), jnp.float32),
            pltpu.VMEM((CHUNK,), jnp.float32),
            pltpu.VMEM((CHUNK,), jnp.float32),
            pltpu.VMEM((CHUNK,), jnp.int32),
            pltpu.VMEM((ROWS, 16), jnp.float32),
            pltpu.VMEM((ROWS, 16), jnp.float32),
            pltpu.VMEM((16,), jnp.int32),
            pltpu.VMEM((16, 16), jnp.int32),
            pltpu.VMEM((7, 128), jnp.int32),
            pltpu.VMEM((7, 128), jnp.int32),
            pltpu.VMEM_SHARED((2, 16, 16), jnp.int32),
            pltpu.SemaphoreType.DMA,
        ],
    )
    def sel(xs_hbm, ys_hbm, zs_hbm, kb_hbm, out_hbm,
            xv, yv, zv, kv, blt, beq, pubv, rbv, ilt, ieq, shared, sem):
        sid = lax.axis_index("s")
        base = sid * CHUNK
        pltpu.sync_copy(xs_hbm.at[pl.ds(base, CHUNK)], xv)
        pltpu.sync_copy(ys_hbm.at[pl.ds(base, CHUNK)], yv)
        pltpu.sync_copy(zs_hbm.at[pl.ds(base, CHUNK)], zv)
        pltpu.sync_copy(kb_hbm.at[pl.ds(base, CHUNK)], kv)
        iota = lax.iota(jnp.int32, 16)

        def publish_sum(slab, vec16):
            pubv[...] = vec16
            pltpu.sync_copy(pubv, shared.at[slab, sid])
            plsc.subcore_barrier()
            pltpu.sync_copy(shared.at[slab], rbv)
            tot = rbv[0]
            for r in range(1, 16):
                tot = tot + rbv[r]
            return tot

        def count_le(t):
            def body(i, acc):
                m = kv[pl.ds(i * 16, 16)] <= t
                return acc + jnp.where(m, 1, 0)
            return lax.fori_loop(0, NV, body, jnp.zeros((16,), jnp.int32))

        lo = jnp.int32(0)
        hi = jnp.int32(0x7F800000)
        for r in range(31):
            mid = lo + ((hi - lo) >> 1)
            cnt = jnp.sum(publish_sum(r & 1, count_le(mid)))
            ok = cnt >= PC
            lo = jnp.where(ok, lo, mid + 1)
            hi = jnp.where(ok, mid, hi)
        kbits = lo

        def cc_body(i, carry):
            olt, oeq = carry
            v = kv[pl.ds(i * 16, 16)]
            x = xv[pl.ds(i * 16, 16)]
            y = yv[pl.ds(i * 16, 16)]
            z = zv[pl.ds(i * 16, 16)]
            mlt = v < kbits
            ilt_ = jnp.where(mlt, 1, 0)
            plt = plsc.cumsum(ilt_)
            rlt = olt + (plt - ilt_)
            c0 = jnp.zeros((16,), jnp.int32)
            plsc.store_scatter(blt, [rlt, c0], x, mask=mlt)
            plsc.store_scatter(blt, [rlt, c0 + 1], y, mask=mlt)
            plsc.store_scatter(blt, [rlt, c0 + 2], z, mask=mlt)
            meq = v == kbits
            ieq_ = jnp.where(meq, 1, 0)
            peq = plsc.cumsum(ieq_)
            req = oeq + (peq - ieq_)
            meqw = meq & (req < ROWS)
            plsc.store_scatter(beq, [req, c0], x, mask=meqw)
            plsc.store_scatter(beq, [req, c0 + 1], y, mask=meqw)
            plsc.store_scatter(beq, [req, c0 + 2], z, mask=meqw)
            return olt + jnp.max(plt), oeq + jnp.max(peq)

        n_lt, n_eq = lax.fori_loop(
            0, NV, cc_body, (jnp.int32(0), jnp.int32(0)))

        v_lt = publish_sum(1, jnp.where(iota == sid, n_lt, 0))
        v_eq = publish_sum(0, jnp.where(iota == sid, n_eq, 0))
        lt_before = jnp.sum(jnp.where(iota < sid, v_lt, 0))
        eq_before = jnp.sum(jnp.where(iota < sid, v_eq, 0))
        total_lt = jnp.sum(v_lt)
        need_eq = PC - total_lt
        take_eq = jnp.clip(need_eq - eq_before, 0, n_eq)
        p_lt = lt_before
        p_eq = total_lt + jnp.minimum(eq_before, need_eq)

        def idx_body(j, _):
            r = j * 16 + iota
            ilt[j >> 3, pl.ds((j & 7) * 16, 16)] = jnp.where(
                r < n_lt, p_lt + r, TRASH)
            ieq[j >> 3, pl.ds((j & 7) * 16, 16)] = jnp.where(
                r < take_eq, p_eq + r, TRASH)
            return 0

        lax.fori_loop(0, ROWS // 16, idx_body, 0)

        for j in range(ROWS // 128):
            @pl.when(n_lt > j * 128)
            def _scat_lt(j=j):
                pltpu.async_copy(
                    blt.at[pl.ds(j * 128, 128)],
                    out_hbm.at[ilt.at[j]], sem).wait()

            @pl.when(take_eq > j * 128)
            def _scat_eq(j=j):
                pltpu.async_copy(
                    beq.at[pl.ds(j * 128, 128)],
                    out_hbm.at[ieq.at[j]], sem).wait()

    return sel(xs, ys, zs, keys)


def _tc_repair(q8, rt):
    def body(q_ref, rt_ref, o_ref):
        q = q_ref[...]
        rt = rt_ref[...]
        g = jnp.dot(q, rt, preferred_element_type=jnp.float32)
        q2 = jnp.sum(q * q, axis=1, keepdims=True)
        r2 = jnp.sum(rt * rt, axis=0, keepdims=True)
        m2 = jnp.min(r2 - 2.0 * g, axis=1, keepdims=True)
        nd = jnp.sqrt(jnp.maximum(q2 + m2, 0.0))
        d = jnp.sqrt(q2) - 0.5
        o_ref[...] = jnp.minimum(nd * jnp.sign(d), d)

    return pl.pallas_call(
        body,
        grid=(N // BM,),
        in_specs=[
            pl.BlockSpec((BM, 8), lambda i: (i, 0)),
            pl.BlockSpec((8, RCOLS), lambda i: (0, 0)),
        ],
        out_specs=pl.BlockSpec((BM, 1), lambda i: (i, 0)),
        out_shape=jax.ShapeDtypeStruct((N, 1), jnp.float32),
    )(q8, rt)


def kernel(query):
    q = query.astype(jnp.float32)
    pad = jnp.full((NPAD - N, 3), PAD_COORD, jnp.float32)
    qp = jnp.concatenate([q, pad], axis=0)
    xs = qp[:, 0]
    ys = qp[:, 1]
    zs = qp[:, 2]
    keys = _tc_keys(xs, ys, zs).reshape(NPAD)
    r16 = _sc_select(xs, ys, zs, keys)
    reps = r16[:PC, :3]
    rt = jnp.zeros((8, RCOLS), jnp.float32)
    rt = rt.at[:3, :PC].set(reps.T)
    rt = rt.at[:3, PC:].set(1e17)
    q8 = jnp.pad(q, ((0, 0), (0, 5)))
    out = _tc_repair(q8, rt)
    return out[:, 0]


# speedup vs baseline: 1.6578x; 1.1623x over previous
import functools

import jax
import jax.numpy as jnp
from jax import lax
from jax.experimental import pallas as pl
from jax.experimental.pallas import tpu as pltpu
from jax.experimental.pallas import tpu_sc as plsc

N = 100000
PC = 800
NPAD = 102400
CHUNK = NPAD // 16
NV = CHUNK // 16
ROWS = 896
RTOT = 904
TRASH = 896
FAR = 1e17
PAD_COORD = 1e18
KR = NPAD // 128
BM = 2048


def _tc_keys(xs, ys, zs):
    def body(x_ref, y_ref, z_ref, k_ref, q_ref):
        x = x_ref[...]
        y = y_ref[...]
        z = z_ref[...]
        a = jnp.abs(jnp.sqrt(x * x + y * y + z * z) - 0.5)
        k_ref[...] = lax.bitcast_convert_type(a, jnp.int32)
        q_ref[0:1, :] = x.reshape(1, 1024)
        q_ref[1:2, :] = y.reshape(1, 1024)
        q_ref[2:3, :] = z.reshape(1, 1024)
        q_ref[3:4, :] = jnp.full((1, 1024), -0.5, jnp.float32)
        q_ref[4:16, :] = jnp.zeros((12, 1024), jnp.float32)

    return pl.pallas_call(
        body,
        grid=(KR // 8,),
        in_specs=[pl.BlockSpec((8, 128), lambda i: (i, 0))] * 3,
        out_specs=[
            pl.BlockSpec((8, 128), lambda i: (i, 0)),
            pl.BlockSpec((16, 1024), lambda i: (0, i)),
        ],
        out_shape=[
            jax.ShapeDtypeStruct((KR, 128), jnp.int32),
            jax.ShapeDtypeStruct((16, NPAD), jnp.float32),
        ],
    )(xs.reshape(KR, 128), ys.reshape(KR, 128), zs.reshape(KR, 128))


def _sc_select(xs, ys, zs, keys):
    mesh = plsc.VectorSubcoreMesh(
        core_axis_name="c", subcore_axis_name="s", num_cores=1)

    @functools.partial(
        pl.kernel,
        out_type=jax.ShapeDtypeStruct((RTOT, 16), jnp.float32),
        mesh=mesh,
        compiler_params=pltpu.CompilerParams(
            needs_layout_passes=False, use_tc_tiling_on_sc=False),
        scratch_types=[
            pltpu.VMEM((CHUNK,), jnp.float32),
            pltpu.VMEM((CHUNK,), jnp.float32),
            pltpu.VMEM((CHUNK,), jnp.float32),
            pltpu.VMEM((CHUNK,), jnp.int32),
            pltpu.VMEM((ROWS, 16), jnp.float32),
            pltpu.VMEM((ROWS, 16), jnp.float32),
            pltpu.VMEM((96, 16), jnp.float32),
            pltpu.VMEM((16,), jnp.int32),
            pltpu.VMEM((16, 16), jnp.int32),
            pltpu.VMEM((7, 128), jnp.int32),
            pltpu.VMEM((7, 128), jnp.int32),
            pltpu.VMEM_SHARED((2, 16, 16), jnp.int32),
            pltpu.SemaphoreType.DMA,
        ],
    )
    def sel(xs_hbm, ys_hbm, zs_hbm, kb_hbm, out_hbm,
            xv, yv, zv, kv, blt, beq, farv, pubv, rbv, ilt, ieq,
            shared, sem):
        sid = lax.axis_index("s")
        base = sid * CHUNK
        pltpu.sync_copy(xs_hbm.at[pl.ds(base, CHUNK)], xv)
        pltpu.sync_copy(ys_hbm.at[pl.ds(base, CHUNK)], yv)
        pltpu.sync_copy(zs_hbm.at[pl.ds(base, CHUNK)], zv)
        pltpu.sync_copy(kb_hbm.at[pl.ds(base, CHUNK)], kv)
        iota = lax.iota(jnp.int32, 16)

        zv16 = jnp.zeros((16,), jnp.float32)

        def z_body(j, _):
            for k in range(8):
                blt[j * 8 + k] = zv16
                beq[j * 8 + k] = zv16
            return 0

        lax.fori_loop(0, ROWS // 8, z_body, 0)

        farrow = jnp.where(iota < 3, FAR, jnp.where(iota == 3, 3.0 * FAR * FAR, 0.0)).astype(jnp.float32)

        def f_body(j, _):
            farv[j] = farrow
            return 0

        lax.fori_loop(0, 96, f_body, 0)

        @pl.when(sid == 0)
        def _far():
            pltpu.sync_copy(farv, out_hbm.at[pl.ds(PC, 96)])

        def publish_sum(slab, vec16):
            pubv[...] = vec16
            pltpu.sync_copy(pubv, shared.at[slab, sid])
            plsc.subcore_barrier()
            pltpu.sync_copy(shared.at[slab], rbv)
            tot = rbv[0]
            for r in range(1, 16):
                tot = tot + rbv[r]
            return tot

        def count_le(t):
            def body(i, acc):
                m = kv[pl.ds(i * 16, 16)] <= t
                return acc + jnp.where(m, 1, 0)
            return lax.fori_loop(0, NV, body, jnp.zeros((16,), jnp.int32))

        lo = jnp.int32(0)
        hi = jnp.int32(0x7F800000)
        for r in range(31):
            mid = lo + ((hi - lo) >> 1)
            cnt = jnp.sum(publish_sum(r & 1, count_le(mid)))
            ok = cnt >= PC
            lo = jnp.where(ok, lo, mid + 1)
            hi = jnp.where(ok, mid, hi)
        kbits = lo

        def cc_body(i, carry):
            olt, oeq = carry
            v = kv[pl.ds(i * 16, 16)]
            x = xv[pl.ds(i * 16, 16)]
            y = yv[pl.ds(i * 16, 16)]
            z = zv[pl.ds(i * 16, 16)]
            r2 = x * x + y * y + z * z
            mlt = v < kbits
            ilt_ = jnp.where(mlt, 1, 0)
            plt = plsc.cumsum(ilt_)
            rlt = olt + (plt - ilt_)
            c0 = jnp.zeros((16,), jnp.int32)
            plsc.store_scatter(blt, [rlt, c0], x, mask=mlt)
            plsc.store_scatter(blt, [rlt, c0 + 1], y, mask=mlt)
            plsc.store_scatter(blt, [rlt, c0 + 2], z, mask=mlt)
            plsc.store_scatter(blt, [rlt, c0 + 3], r2, mask=mlt)
            meq = v == kbits
            ieq_ = jnp.where(meq, 1, 0)
            peq = plsc.cumsum(ieq_)
            req = oeq + (peq - ieq_)
            meqw = meq & (req < ROWS)
            plsc.store_scatter(beq, [req, c0], x, mask=meqw)
            plsc.store_scatter(beq, [req, c0 + 1], y, mask=meqw)
            plsc.store_scatter(beq, [req, c0 + 2], z, mask=meqw)
            plsc.store_scatter(beq, [req, c0 + 3], r2, mask=meqw)
            return olt + jnp.max(plt), oeq + jnp.max(peq)

        n_lt, n_eq = lax.fori_loop(
            0, NV, cc_body, (jnp.int32(0), jnp.int32(0)))

        v_lt = publish_sum(1, jnp.where(iota == sid, n_lt, 0))
        v_eq = publish_sum(0, jnp.where(iota == sid, n_eq, 0))
        lt_before = jnp.sum(jnp.where(iota < sid, v_lt, 0))
        eq_before = jnp.sum(jnp.where(iota < sid, v_eq, 0))
        total_lt = jnp.sum(v_lt)
        need_eq = PC - total_lt
        take_eq = jnp.clip(need_eq - eq_before, 0, n_eq)
        p_lt = lt_before
        p_eq = total_lt + jnp.minimum(eq_before, need_eq)

        def idx_body(j, _):
            r = j * 16 + iota
            ilt[j >> 3, pl.ds((j & 7) * 16, 16)] = jnp.where(
                r < n_lt, p_lt + r, TRASH)
            ieq[j >> 3, pl.ds((j & 7) * 16, 16)] = jnp.where(
                r < take_eq, p_eq + r, TRASH)
            return 0

        lax.fori_loop(0, ROWS // 16, idx_body, 0)

        for j in range(ROWS // 128):
            @pl.when(n_lt > j * 128)
            def _scat_lt(j=j):
                pltpu.async_copy(
                    blt.at[pl.ds(j * 128, 128)],
                    out_hbm.at[ilt.at[j]], sem).wait()

            @pl.when(take_eq > j * 128)
            def _scat_eq(j=j):
                pltpu.async_copy(
                    beq.at[pl.ds(j * 128, 128)],
                    out_hbm.at[ieq.at[j]], sem).wait()

    return sel(xs, ys, zs, keys)


def _tc_repair(rtab, qt):
    def body(r_ref, q_ref, o_ref):
        rt = r_ref[...]
        q = q_ref[...]
        g = jnp.dot(rt, q, preferred_element_type=jnp.float32)
        mx = jnp.max(g, axis=0, keepdims=True)
        q2 = (q[0:1, :] * q[0:1, :] + q[1:2, :] * q[1:2, :]
              + q[2:3, :] * q[2:3, :])
        nd = jnp.sqrt(jnp.maximum(q2 - 2.0 * mx, 0.0))
        d = jnp.sqrt(q2) - 0.5
        res = jnp.minimum(nd * jnp.sign(d), d)
        o_ref[...] = res.reshape(BM)

    return pl.pallas_call(
        body,
        grid=(NPAD // BM,),
        in_specs=[
            pl.BlockSpec((ROWS, 16), lambda i: (0, 0)),
            pl.BlockSpec((16, BM), lambda i: (0, i)),
        ],
        out_specs=pl.BlockSpec((BM,), lambda i: (i,)),
        out_shape=jax.ShapeDtypeStruct((NPAD,), jnp.float32),
    )(rtab, qt)


def kernel(query):
    q = query.astype(jnp.float32)
    pad = jnp.full((NPAD - N, 3), PAD_COORD, jnp.float32)
    qp = jnp.concatenate([q, pad], axis=0)
    xs = qp[:, 0]
    ys = qp[:, 1]
    zs = qp[:, 2]
    keys, qt = _tc_keys(xs, ys, zs)
    rtab = _sc_select(xs, ys, zs, keys.reshape(NPAD))
    out = _tc_repair(rtab, qt)
    return out[:N]


# speedup vs baseline: 2.0130x; 1.2143x over previous
import functools

import jax
import jax.numpy as jnp
from jax import lax
from jax.experimental import pallas as pl
from jax.experimental.pallas import tpu as pltpu
from jax.experimental.pallas import tpu_sc as plsc

N = 100000
PC = 800
NPAD = 102400
CHUNK = NPAD // 16
NV = CHUNK // 16
ROWS = 896
RTOT = 904
TRASH = 896
FAR = 1e17
PAD_COORD = 1e18
KR = NPAD // 128
BM = 2048


def _tc_keys(xs, ys, zs):
    def body(x_ref, y_ref, z_ref, k_ref, q_ref):
        x = x_ref[...]
        y = y_ref[...]
        z = z_ref[...]
        a = jnp.abs(jnp.sqrt(x * x + y * y + z * z) - 0.5)
        k_ref[...] = lax.bitcast_convert_type(a, jnp.int32)
        q_ref[0:1, :] = x.reshape(1, 1024)
        q_ref[1:2, :] = y.reshape(1, 1024)
        q_ref[2:3, :] = z.reshape(1, 1024)
        q_ref[3:4, :] = jnp.full((1, 1024), -0.5, jnp.float32)
        q_ref[4:16, :] = jnp.zeros((12, 1024), jnp.float32)

    return pl.pallas_call(
        body,
        grid=(KR // 8,),
        in_specs=[pl.BlockSpec((8, 128), lambda i: (i, 0))] * 3,
        out_specs=[
            pl.BlockSpec((8, 128), lambda i: (i, 0)),
            pl.BlockSpec((16, 1024), lambda i: (0, i)),
        ],
        out_shape=[
            jax.ShapeDtypeStruct((KR, 128), jnp.int32),
            jax.ShapeDtypeStruct((16, NPAD), jnp.float32),
        ],
    )(xs.reshape(KR, 128), ys.reshape(KR, 128), zs.reshape(KR, 128))


def _sc_select(xs, ys, zs, keys):
    mesh = plsc.VectorSubcoreMesh(
        core_axis_name="c", subcore_axis_name="s", num_cores=1)

    @functools.partial(
        pl.kernel,
        out_type=jax.ShapeDtypeStruct((RTOT, 16), jnp.float32),
        mesh=mesh,
        compiler_params=pltpu.CompilerParams(
            needs_layout_passes=False, use_tc_tiling_on_sc=False),
        scratch_types=[
            pltpu.VMEM((CHUNK,), jnp.float32),
            pltpu.VMEM((CHUNK,), jnp.float32),
            pltpu.VMEM((CHUNK,), jnp.float32),
            pltpu.VMEM((CHUNK,), jnp.int32),
            pltpu.VMEM((ROWS, 16), jnp.float32),
            pltpu.VMEM((ROWS, 16), jnp.float32),
            pltpu.VMEM((96, 16), jnp.float32),
            pltpu.VMEM((16,), jnp.int32),
            pltpu.VMEM((16, 16), jnp.int32),
            pltpu.VMEM((7, 128), jnp.int32),
            pltpu.VMEM((7, 128), jnp.int32),
            pltpu.VMEM_SHARED((2, 16, 16), jnp.int32),
            pltpu.SemaphoreType.DMA,
        ],
    )
    def sel(xs_hbm, ys_hbm, zs_hbm, kb_hbm, out_hbm,
            xv, yv, zv, kv, blt, beq, farv, pubv, rbv, ilt, ieq,
            shared, sem):
        sid = lax.axis_index("s")
        base = sid * CHUNK
        d1 = pltpu.async_copy(xs_hbm.at[pl.ds(base, CHUNK)], xv, sem)
        d2 = pltpu.async_copy(ys_hbm.at[pl.ds(base, CHUNK)], yv, sem)
        d3 = pltpu.async_copy(zs_hbm.at[pl.ds(base, CHUNK)], zv, sem)
        d4 = pltpu.async_copy(kb_hbm.at[pl.ds(base, CHUNK)], kv, sem)
        d1.wait()
        d2.wait()
        d3.wait()
        d4.wait()
        iota = lax.iota(jnp.int32, 16)

        zv16 = jnp.zeros((16,), jnp.float32)

        def z_body(j, _):
            for k in range(8):
                blt[j * 8 + k] = zv16
                beq[j * 8 + k] = zv16
            return 0

        lax.fori_loop(0, ROWS // 8, z_body, 0)

        farrow = jnp.where(iota < 3, FAR, jnp.where(iota == 3, 3.0 * FAR * FAR, 0.0)).astype(jnp.float32)

        def f_body(j, _):
            farv[j] = farrow
            return 0

        lax.fori_loop(0, 96, f_body, 0)

        @pl.when(sid == 0)
        def _far():
            pltpu.sync_copy(farv, out_hbm.at[pl.ds(PC, 96)])

        def publish_sum(slab, vec16):
            pubv[...] = vec16
            pltpu.sync_copy(pubv, shared.at[slab, sid])
            plsc.subcore_barrier()
            pltpu.sync_copy(shared.at[slab], rbv)
            tot = rbv[0]
            for r in range(1, 16):
                tot = tot + rbv[r]
            return tot

        def count_le(t):
            def body(i, acc):
                b0 = i * 256
                for k in range(16):
                    m = kv[pl.ds(b0 + k * 16, 16)] <= t
                    acc = acc + jnp.where(m, 1, 0)
                return acc
            return lax.fori_loop(
                0, NV // 16, body, jnp.zeros((16,), jnp.int32))

        lo = jnp.int32(0)
        hi = jnp.int32(0x7F800000)
        for r in range(31):
            mid = lo + ((hi - lo) >> 1)
            cnt = jnp.sum(publish_sum(r & 1, count_le(mid)))
            ok = cnt >= PC
            lo = jnp.where(ok, lo, mid + 1)
            hi = jnp.where(ok, mid, hi)
        kbits = lo

        def cc_body(i, carry):
            olt, oeq = carry
            v = kv[pl.ds(i * 16, 16)]
            x = xv[pl.ds(i * 16, 16)]
            y = yv[pl.ds(i * 16, 16)]
            z = zv[pl.ds(i * 16, 16)]
            r2 = x * x + y * y + z * z
            mlt = v < kbits
            ilt_ = jnp.where(mlt, 1, 0)
            plt = plsc.cumsum(ilt_)
            rlt = olt + (plt - ilt_)
            c0 = jnp.zeros((16,), jnp.int32)
            plsc.store_scatter(blt, [rlt, c0], x, mask=mlt)
            plsc.store_scatter(blt, [rlt, c0 + 1], y, mask=mlt)
            plsc.store_scatter(blt, [rlt, c0 + 2], z, mask=mlt)
            plsc.store_scatter(blt, [rlt, c0 + 3], r2, mask=mlt)
            meq = v == kbits
            ieq_ = jnp.where(meq, 1, 0)
            peq = plsc.cumsum(ieq_)
            req = oeq + (peq - ieq_)
            meqw = meq & (req < ROWS)
            plsc.store_scatter(beq, [req, c0], x, mask=meqw)
            plsc.store_scatter(beq, [req, c0 + 1], y, mask=meqw)
            plsc.store_scatter(beq, [req, c0 + 2], z, mask=meqw)
            plsc.store_scatter(beq, [req, c0 + 3], r2, mask=meqw)
            return olt + jnp.max(plt), oeq + jnp.max(peq)

        n_lt, n_eq = lax.fori_loop(
            0, NV, cc_body, (jnp.int32(0), jnp.int32(0)))

        v_lt = publish_sum(1, jnp.where(iota == sid, n_lt, 0))
        v_eq = publish_sum(0, jnp.where(iota == sid, n_eq, 0))
        lt_before = jnp.sum(jnp.where(iota < sid, v_lt, 0))
        eq_before = jnp.sum(jnp.where(iota < sid, v_eq, 0))
        total_lt = jnp.sum(v_lt)
        need_eq = PC - total_lt
        take_eq = jnp.clip(need_eq - eq_before, 0, n_eq)
        p_lt = lt_before
        p_eq = total_lt + jnp.minimum(eq_before, need_eq)

        def idx_body(j, _):
            r = j * 16 + iota
            ilt[j >> 3, pl.ds((j & 7) * 16, 16)] = jnp.where(
                r < n_lt, p_lt + r, TRASH)
            ieq[j >> 3, pl.ds((j & 7) * 16, 16)] = jnp.where(
                r < take_eq, p_eq + r, TRASH)
            return 0

        lax.fori_loop(0, ROWS // 16, idx_body, 0)

        for j in range(ROWS // 128):
            @pl.when(n_lt > j * 128)
            def _scat_lt(j=j):
                pltpu.async_copy(
                    blt.at[pl.ds(j * 128, 128)],
                    out_hbm.at[ilt.at[j]], sem).wait()

            @pl.when(take_eq > j * 128)
            def _scat_eq(j=j):
                pltpu.async_copy(
                    beq.at[pl.ds(j * 128, 128)],
                    out_hbm.at[ieq.at[j]], sem).wait()

    return sel(xs, ys, zs, keys)


def _tc_repair(rtab, qt):
    def body(r_ref, q_ref, o_ref):
        rt = r_ref[...]
        q = q_ref[...]
        g = jnp.dot(rt, q, preferred_element_type=jnp.float32)
        mx = jnp.max(g, axis=0, keepdims=True)
        q2 = (q[0:1, :] * q[0:1, :] + q[1:2, :] * q[1:2, :]
              + q[2:3, :] * q[2:3, :])
        nd = jnp.sqrt(jnp.maximum(q2 - 2.0 * mx, 0.0))
        d = jnp.sqrt(q2) - 0.5
        res = jnp.minimum(nd * jnp.sign(d), d)
        o_ref[...] = res.reshape(BM)

    return pl.pallas_call(
        body,
        grid=(NPAD // BM,),
        in_specs=[
            pl.BlockSpec((ROWS, 16), lambda i: (0, 0)),
            pl.BlockSpec((16, BM), lambda i: (0, i)),
        ],
        out_specs=pl.BlockSpec((BM,), lambda i: (i,)),
        out_shape=jax.ShapeDtypeStruct((NPAD,), jnp.float32),
    )(rtab, qt)


def kernel(query):
    q = query.astype(jnp.float32)
    pad = jnp.full((NPAD - N, 3), PAD_COORD, jnp.float32)
    qp = jnp.concatenate([q, pad], axis=0)
    xs = qp[:, 0]
    ys = qp[:, 1]
    zs = qp[:, 2]
    keys, qt = _tc_keys(xs, ys, zs)
    rtab = _sc_select(xs, ys, zs, keys.reshape(NPAD))
    out = _tc_repair(rtab, qt)
    return out[:N]
